# Initial kernel scaffold; baseline (speedup 1.0000x reference)
import jax
import jax.numpy as jnp
from jax.experimental import pallas as pl


def kernel(embeddings, labels, prototypes, initialized):
    raise NotImplementedError("write your pallas kernel here")


# speedup vs baseline: 2.6470x; 2.6470x over previous
import functools

import jax
import jax.numpy as jnp
from jax import lax
from jax.experimental import pallas as pl
from jax.experimental.pallas import tpu as pltpu
from jax.experimental.pallas import tpu_sc as plsc

_B, _D, _C = 16384, 128, 100
_MOM = 0.9

_NC, _NS = 2, 16
_NW = _NC * _NS
_RPW = _B // _NW
_CHUNK = 128
_NCHUNK = _RPW // _CHUNK
_CPAD = 128
_ZROWS = _CPAD // _NS


def _sc_segsum_body(emb_hbm, lab_hbm, psums_hbm, pcnts_hbm,
                    emb_v, idx_v, ones_v, zrow_v, sums_sh, cnts_sh):
    cid = lax.axis_index("c")
    sid = lax.axis_index("s")

    zero16 = jnp.zeros((16,), jnp.float32)
    one16 = jnp.ones((16,), jnp.float32)

    def _fill_ones(i, c):
        for j in range(_D // 16):
            ones_v[i, pl.ds(j * 16, 16)] = one16
        return c

    lax.fori_loop(0, _CHUNK, _fill_ones, 0)

    def _fill_zero(i, c):
        for j in range(_D // 16):
            zrow_v[i, pl.ds(j * 16, 16)] = zero16
        return c

    lax.fori_loop(0, _ZROWS, _fill_zero, 0)

    pltpu.sync_copy(zrow_v, sums_sh.at[pl.ds(sid * _ZROWS, _ZROWS)])
    pltpu.sync_copy(zrow_v, cnts_sh.at[pl.ds(sid * _ZROWS, _ZROWS)])
    plsc.subcore_barrier()

    base = (cid * _NS + sid) * _RPW
    for k in range(_NCHUNK):
        off = base + k * _CHUNK
        pltpu.sync_copy(lab_hbm.at[pl.ds(off, _CHUNK)], idx_v)
        pltpu.sync_copy(emb_hbm.at[pl.ds(off, _CHUNK)], emb_v)
        pltpu.sync_copy(emb_v, sums_sh.at[idx_v], add=True)
        pltpu.sync_copy(ones_v, cnts_sh.at[idx_v], add=True)
    plsc.subcore_barrier()

    r0 = sid * _ZROWS
    pltpu.sync_copy(sums_sh.at[pl.ds(r0, _ZROWS)],
                    psums_hbm.at[cid, pl.ds(r0, _ZROWS)])
    pltpu.sync_copy(cnts_sh.at[pl.ds(r0, _ZROWS)],
                    pcnts_hbm.at[cid, pl.ds(r0, _ZROWS)])


@functools.cache
def _sc_segsum():
    return pl.kernel(
        _sc_segsum_body,
        out_type=(jax.ShapeDtypeStruct((_NC, _CPAD, _D), jnp.float32),
                  jax.ShapeDtypeStruct((_NC, _CPAD, _D), jnp.float32)),
        mesh=plsc.VectorSubcoreMesh(core_axis_name="c", subcore_axis_name="s",
                                    num_cores=_NC, num_subcores=_NS),
        scratch_types=[
            pltpu.VMEM((_CHUNK, _D), jnp.float32),
            pltpu.VMEM((_CHUNK,), jnp.int32),
            pltpu.VMEM((_CHUNK, _D), jnp.float32),
            pltpu.VMEM((_ZROWS, _D), jnp.float32),
            pltpu.VMEM_SHARED((_CPAD, _D), jnp.float32),
            pltpu.VMEM_SHARED((_CPAD, _D), jnp.float32),
        ],
    )

_BLK = 2048


def _tc_body(emb_ref, psums_ref, pcnt_ref, proto_ref, init_ref,
             newp_ref, pred_ref, dist_ref):
    sums = psums_ref[0, :_C] + psums_ref[1, :_C]
    cnt2 = pcnt_ref[0, :_C] + pcnt_ref[1, :_C]
    cnt = cnt2[:, 0:1]
    cls_mean = sums / jnp.maximum(cnt, 1.0)
    present = cnt > 0.0
    initm = init_ref[...] > 0.0
    protos = proto_ref[...]
    ema = _MOM * protos + (1.0 - _MOM) * cls_mean
    upd = jnp.where(initm, ema, cls_mean)
    newp = jnp.where(present, upd, protos)
    newp_ref[...] = newp
    new_init = jnp.logical_or(initm, present)

    pn = jnp.sqrt(jnp.sum(newp * newp, axis=1, keepdims=True))
    pnorm = newp / jnp.maximum(pn, 1e-12)

    e = emb_ref[...]
    en = jnp.sqrt(jnp.sum(e * e, axis=1, keepdims=True))
    en_inv = e / jnp.maximum(en, 1e-12)

    simsT = lax.dot_general(pnorm, en_inv, (((1,), (1,)), ((), ())),
                            preferred_element_type=jnp.float32)
    simsT = jnp.where(new_init, simsT, -jnp.inf)
    m = jnp.max(simsT, axis=0, keepdims=True)
    row = lax.broadcasted_iota(jnp.int32, simsT.shape, 0)
    pred = jnp.min(jnp.where(simsT == m, row, _C), axis=0, keepdims=True)
    pred_ref[0] = pred
    dist_ref[0] = 1.0 - m


_tc_predict = pl.pallas_call(
    _tc_body,
    grid=(_B // _BLK,),
    in_specs=[
        pl.BlockSpec((_BLK, _D), lambda i: (i, 0)),
        pl.BlockSpec((_NC, _CPAD, _D), lambda i: (0, 0, 0)),
        pl.BlockSpec((_NC, _CPAD, _D), lambda i: (0, 0, 0)),
        pl.BlockSpec((_C, _D), lambda i: (0, 0)),
        pl.BlockSpec((_C, 1), lambda i: (0, 0)),
    ],
    out_specs=[
        pl.BlockSpec((_C, _D), lambda i: (0, 0)),
        pl.BlockSpec((1, 1, _BLK), lambda i: (i, 0, 0)),
        pl.BlockSpec((1, 1, _BLK), lambda i: (i, 0, 0)),
    ],
    out_shape=[
        jax.ShapeDtypeStruct((_C, _D), jnp.float32),
        jax.ShapeDtypeStruct((_B // _BLK, 1, _BLK), jnp.int32),
        jax.ShapeDtypeStruct((_B // _BLK, 1, _BLK), jnp.float32),
    ],
)


def kernel(embeddings, labels, prototypes, initialized):
    psums, pcnts = _sc_segsum()(embeddings, labels)
    init_col = initialized.astype(jnp.float32).reshape(_C, 1)
    newp, pred2d, dist2d = _tc_predict(embeddings, psums, pcnts,
                                       prototypes, init_col)
    return newp, pred2d.reshape(_B), dist2d.reshape(_B)


# speedup vs baseline: 2.6602x; 1.0050x over previous
import functools

import jax
import jax.numpy as jnp
from jax import lax
from jax.experimental import pallas as pl
from jax.experimental.pallas import tpu as pltpu
from jax.experimental.pallas import tpu_sc as plsc

_B, _D, _C = 16384, 128, 100
_MOM = 0.9

_NC, _NS = 2, 16
_NW = _NC * _NS
_RPW = _B // _NW
_CHUNK = 128
_NCHUNK = _RPW // _CHUNK
_CPAD = 128
_ZROWS = _CPAD // _NS


def _sc_segsum_body(emb_hbm, lab_hbm, psums_hbm,
                    emb_v, idx_v, zrow_v, sums_sh):
    cid = lax.axis_index("c")
    sid = lax.axis_index("s")

    zero16 = jnp.zeros((16,), jnp.float32)

    def _fill_zero(i, c):
        for j in range(_D // 16):
            zrow_v[i, pl.ds(j * 16, 16)] = zero16
        return c

    lax.fori_loop(0, _ZROWS, _fill_zero, 0)

    pltpu.sync_copy(zrow_v, sums_sh.at[pl.ds(sid * _ZROWS, _ZROWS)])
    plsc.subcore_barrier()

    base = (cid * _NS + sid) * _RPW
    for k in range(_NCHUNK):
        off = base + k * _CHUNK
        pltpu.sync_copy(lab_hbm.at[pl.ds(off, _CHUNK)], idx_v)
        pltpu.sync_copy(emb_hbm.at[pl.ds(off, _CHUNK)], emb_v)
        pltpu.sync_copy(emb_v, sums_sh.at[idx_v], add=True)
    plsc.subcore_barrier()

    r0 = sid * _ZROWS
    pltpu.sync_copy(sums_sh.at[pl.ds(r0, _ZROWS)],
                    psums_hbm.at[cid, pl.ds(r0, _ZROWS)])


@functools.cache
def _sc_segsum():
    return pl.kernel(
        _sc_segsum_body,
        out_type=jax.ShapeDtypeStruct((_NC, _CPAD, _D), jnp.float32),
        mesh=plsc.VectorSubcoreMesh(core_axis_name="c", subcore_axis_name="s",
                                    num_cores=_NC, num_subcores=_NS),
        scratch_types=[
            pltpu.VMEM((_CHUNK, _D), jnp.float32),
            pltpu.VMEM((_CHUNK,), jnp.int32),
            pltpu.VMEM((_ZROWS, _D), jnp.float32),
            pltpu.VMEM_SHARED((_CPAD, _D), jnp.float32),
        ],
    )


_BLK = 2048
_NBLK = _B // _BLK


def _tc_body(emb_ref, lab_ref, psums_ref, proto_ref, init_ref,
             newp_ref, pred_ref, dist_ref, cnt_s):
    i = pl.program_id(0)

    @pl.when(i == 0)
    def _():
        cnt_s[...] = jnp.zeros((_CPAD, 1), jnp.float32)

    @pl.when(i < _NBLK)
    def _():
        lab = lab_ref[0]
        oh = (jnp.broadcast_to(lab, (_CPAD, _BLK))
              == lax.broadcasted_iota(jnp.int32, (_CPAD, _BLK), 0))
        cnt_s[...] += jnp.sum(oh.astype(jnp.float32), axis=1, keepdims=True)

    @pl.when(i >= _NBLK)
    def _():
        sums = psums_ref[0, :_C] + psums_ref[1, :_C]
        cnt = cnt_s[...][:_C]
        cls_mean = sums / jnp.maximum(cnt, 1.0)
        present = cnt > 0.0
        initm = init_ref[...] > 0.0
        protos = proto_ref[...]
        ema = _MOM * protos + (1.0 - _MOM) * cls_mean
        upd = jnp.where(initm, ema, cls_mean)
        newp = jnp.where(present, upd, protos)
        newp_ref[...] = newp
        new_init = jnp.logical_or(initm, present)

        pn = jnp.sqrt(jnp.sum(newp * newp, axis=1, keepdims=True))
        pnorm = newp / jnp.maximum(pn, 1e-12)

        e = emb_ref[...]
        en = jnp.sqrt(jnp.sum(e * e, axis=1, keepdims=True))
        en_inv = e / jnp.maximum(en, 1e-12)

        simsT = lax.dot_general(pnorm, en_inv, (((1,), (1,)), ((), ())),
                                preferred_element_type=jnp.float32)
        simsT = jnp.where(new_init, simsT, -jnp.inf)
        m = jnp.max(simsT, axis=0, keepdims=True)
        row = lax.broadcasted_iota(jnp.int32, simsT.shape, 0)
        pred = jnp.min(jnp.where(simsT == m, row, _C), axis=0, keepdims=True)
        pred_ref[0] = pred
        dist_ref[0] = 1.0 - m


_tc_predict = pl.pallas_call(
    _tc_body,
    grid=(2 * _NBLK,),
    in_specs=[
        pl.BlockSpec((_BLK, _D), lambda i: (jnp.maximum(i - _NBLK, 0), 0)),
        pl.BlockSpec((1, 1, _BLK), lambda i: (jnp.minimum(i, _NBLK - 1), 0, 0)),
        pl.BlockSpec((_NC, _CPAD, _D), lambda i: (0, 0, 0)),
        pl.BlockSpec((_C, _D), lambda i: (0, 0)),
        pl.BlockSpec((_C, 1), lambda i: (0, 0)),
    ],
    out_specs=[
        pl.BlockSpec((_C, _D), lambda i: (0, 0)),
        pl.BlockSpec((1, 1, _BLK), lambda i: (jnp.maximum(i - _NBLK, 0), 0, 0)),
        pl.BlockSpec((1, 1, _BLK), lambda i: (jnp.maximum(i - _NBLK, 0), 0, 0)),
    ],
    out_shape=[
        jax.ShapeDtypeStruct((_C, _D), jnp.float32),
        jax.ShapeDtypeStruct((_NBLK, 1, _BLK), jnp.int32),
        jax.ShapeDtypeStruct((_NBLK, 1, _BLK), jnp.float32),
    ],
    scratch_shapes=[pltpu.VMEM((_CPAD, 1), jnp.float32)],
)


def kernel(embeddings, labels, prototypes, initialized):
    psums = _sc_segsum()(embeddings, labels)
    lab3 = labels.reshape(_NBLK, 1, _BLK)
    init_col = initialized.astype(jnp.float32).reshape(_C, 1)
    newp, pred2d, dist2d = _tc_predict(embeddings, lab3, psums,
                                       prototypes, init_col)
    return newp, pred2d.reshape(_B), dist2d.reshape(_B)


# speedup vs baseline: 3.2529x; 1.2228x over previous
import functools

import jax
import jax.numpy as jnp
from jax import lax
from jax.experimental import pallas as pl
from jax.experimental.pallas import tpu as pltpu
from jax.experimental.pallas import tpu_sc as plsc

_B, _D, _C = 16384, 128, 100
_MOM = 0.9

_NC, _NS = 2, 16
_NW = _NC * _NS
_RPW = _B // _NW
_CHUNK = 128
_NCHUNK = _RPW // _CHUNK
_CPAD = 128
_ZROWS = _CPAD // _NS


def _sc_segsum_body(emb_hbm, lab_hbm, psums_hbm,
                    emb_v0, emb_v1, idx_v0, idx_v1, zrow_v, sums_sh,
                    sem0, sem1):
    cid = lax.axis_index("c")
    sid = lax.axis_index("s")
    base = (cid * _NS + sid) * _RPW

    embs = (emb_v0, emb_v1)
    idxs = (idx_v0, idx_v1)
    sems = (sem0, sem1)

    def _issue(k):
        off = base + k * _CHUNK
        s = sems[k % 2]
        d1 = pltpu.async_copy(lab_hbm.at[pl.ds(off, _CHUNK)], idxs[k % 2], s)
        d2 = pltpu.async_copy(emb_hbm.at[pl.ds(off, _CHUNK)], embs[k % 2], s)
        return d1, d2

    descs = [None, None]
    descs[0] = _issue(0)

    zero16 = jnp.zeros((16,), jnp.float32)

    def _fill_zero(i, c):
        for j in range(_D // 16):
            zrow_v[i, pl.ds(j * 16, 16)] = zero16
        return c

    lax.fori_loop(0, _ZROWS, _fill_zero, 0)

    pltpu.sync_copy(zrow_v, sums_sh.at[pl.ds(sid * _ZROWS, _ZROWS)])
    plsc.subcore_barrier()

    for k in range(_NCHUNK):
        if k + 1 < _NCHUNK:
            descs[(k + 1) % 2] = _issue(k + 1)
        d1, d2 = descs[k % 2]
        d1.wait()
        d2.wait()
        pltpu.sync_copy(embs[k % 2], sums_sh.at[idxs[k % 2]], add=True)
    plsc.subcore_barrier()

    r0 = sid * _ZROWS
    pltpu.sync_copy(sums_sh.at[pl.ds(r0, _ZROWS)],
                    psums_hbm.at[cid, pl.ds(r0, _ZROWS)])


@functools.cache
def _sc_segsum():
    return pl.kernel(
        _sc_segsum_body,
        out_type=jax.ShapeDtypeStruct((_NC, _CPAD, _D), jnp.float32),
        mesh=plsc.VectorSubcoreMesh(core_axis_name="c", subcore_axis_name="s",
                                    num_cores=_NC, num_subcores=_NS),
        scratch_types=[
            pltpu.VMEM((_CHUNK, _D), jnp.float32),
            pltpu.VMEM((_CHUNK, _D), jnp.float32),
            pltpu.VMEM((_CHUNK,), jnp.int32),
            pltpu.VMEM((_CHUNK,), jnp.int32),
            pltpu.VMEM((_ZROWS, _D), jnp.float32),
            pltpu.VMEM_SHARED((_CPAD, _D), jnp.float32),
            pltpu.SemaphoreType.DMA,
            pltpu.SemaphoreType.DMA,
        ],
    )


_BLK = 2048
_NBLK = _B // _BLK


def _tc_counts_body(lab_ref, cnt_ref):
    i = pl.program_id(0)

    @pl.when(i == 0)
    def _():
        cnt_ref[...] = jnp.zeros((_CPAD, 1), jnp.float32)

    lab = lab_ref[0]
    oh = (jnp.broadcast_to(lab, (_CPAD, _BLK))
          == lax.broadcasted_iota(jnp.int32, (_CPAD, _BLK), 0))
    ones = jnp.ones((_BLK, 1), jnp.float32)
    cnt_ref[...] += lax.dot_general(oh.astype(jnp.float32), ones,
                                    (((1,), (0,)), ((), ())),
                                    preferred_element_type=jnp.float32)


_tc_counts = pl.pallas_call(
    _tc_counts_body,
    grid=(_NBLK,),
    in_specs=[pl.BlockSpec((1, 1, _BLK), lambda i: (i, 0, 0))],
    out_specs=pl.BlockSpec((_CPAD, 1), lambda i: (0, 0)),
    out_shape=jax.ShapeDtypeStruct((_CPAD, 1), jnp.float32),
)


def _tc_body(emb_ref, psums_ref, cnt_ref, proto_ref, init_ref,
             newp_ref, pred_ref, dist_ref):
    sums = psums_ref[0, :_C] + psums_ref[1, :_C]
    cnt = cnt_ref[...][:_C]
    cls_mean = sums / jnp.maximum(cnt, 1.0)
    present = cnt > 0.0
    initm = init_ref[...] > 0.0
    protos = proto_ref[...]
    ema = _MOM * protos + (1.0 - _MOM) * cls_mean
    upd = jnp.where(initm, ema, cls_mean)
    newp = jnp.where(present, upd, protos)
    newp_ref[...] = newp
    new_init = jnp.logical_or(initm, present)

    pn = jnp.sqrt(jnp.sum(newp * newp, axis=1, keepdims=True))
    pnorm = newp / jnp.maximum(pn, 1e-12)

    e = emb_ref[...]
    en = jnp.sqrt(jnp.sum(e * e, axis=1, keepdims=True))
    en_inv = e / jnp.maximum(en, 1e-12)

    simsT = lax.dot_general(pnorm, en_inv, (((1,), (1,)), ((), ())),
                            preferred_element_type=jnp.float32)
    simsT = jnp.where(new_init, simsT, -jnp.inf)
    m = jnp.max(simsT, axis=0, keepdims=True)
    row = lax.broadcasted_iota(jnp.int32, simsT.shape, 0)
    pred = jnp.min(jnp.where(simsT == m, row, _C), axis=0, keepdims=True)
    pred_ref[0] = pred
    dist_ref[0] = 1.0 - m


_tc_predict = pl.pallas_call(
    _tc_body,
    grid=(_NBLK,),
    in_specs=[
        pl.BlockSpec((_BLK, _D), lambda i: (i, 0)),
        pl.BlockSpec((_NC, _CPAD, _D), lambda i: (0, 0, 0)),
        pl.BlockSpec((_CPAD, 1), lambda i: (0, 0)),
        pl.BlockSpec((_C, _D), lambda i: (0, 0)),
        pl.BlockSpec((_C, 1), lambda i: (0, 0)),
    ],
    out_specs=[
        pl.BlockSpec((_C, _D), lambda i: (0, 0)),
        pl.BlockSpec((1, 1, _BLK), lambda i: (i, 0, 0)),
        pl.BlockSpec((1, 1, _BLK), lambda i: (i, 0, 0)),
    ],
    out_shape=[
        jax.ShapeDtypeStruct((_C, _D), jnp.float32),
        jax.ShapeDtypeStruct((_NBLK, 1, _BLK), jnp.int32),
        jax.ShapeDtypeStruct((_NBLK, 1, _BLK), jnp.float32),
    ],
)


def kernel(embeddings, labels, prototypes, initialized):
    psums = _sc_segsum()(embeddings, labels)
    lab3 = labels.reshape(_NBLK, 1, _BLK)
    cnts = _tc_counts(lab3)
    init_col = initialized.astype(jnp.float32).reshape(_C, 1)
    newp, pred2d, dist2d = _tc_predict(embeddings, psums, cnts,
                                       prototypes, init_col)
    return newp, pred2d.reshape(_B), dist2d.reshape(_B)


# speedup vs baseline: 3.2858x; 1.0101x over previous
import functools

import jax
import jax.numpy as jnp
from jax import lax
from jax.experimental import pallas as pl
from jax.experimental.pallas import tpu as pltpu
from jax.experimental.pallas import tpu_sc as plsc

_B, _D, _C = 16384, 128, 100
_MOM = 0.9

_NC, _NS = 2, 16
_NW = _NC * _NS
_RPW = _B // _NW
_CHUNK = 128
_NCHUNK = _RPW // _CHUNK
_CPAD = 128
_ZROWS = _CPAD // _NS


def _sc_segsum_body(emb_hbm, lab_hbm, psums_hbm,
                    emb_v0, emb_v1, emb_v2, emb_v3,
                    idx_v0, idx_v1, idx_v2, idx_v3, zrow_v, sums_sh,
                    sem0, sem1, sem2, sem3):
    cid = lax.axis_index("c")
    sid = lax.axis_index("s")
    base = (cid * _NS + sid) * _RPW

    embs = (emb_v0, emb_v1, emb_v2, emb_v3)
    idxs = (idx_v0, idx_v1, idx_v2, idx_v3)
    sems = (sem0, sem1, sem2, sem3)

    descs = []
    for k in range(_NCHUNK):
        off = base + k * _CHUNK
        descs.append((
            pltpu.async_copy(lab_hbm.at[pl.ds(off, _CHUNK)], idxs[k], sems[k]),
            pltpu.async_copy(emb_hbm.at[pl.ds(off, _CHUNK)], embs[k], sems[k]),
        ))

    zero16 = jnp.zeros((16,), jnp.float32)

    def _fill_zero(i, c):
        for j in range(_D // 16):
            zrow_v[i, pl.ds(j * 16, 16)] = zero16
        return c

    lax.fori_loop(0, _ZROWS, _fill_zero, 0)

    pltpu.sync_copy(zrow_v, sums_sh.at[pl.ds(sid * _ZROWS, _ZROWS)])
    plsc.subcore_barrier()

    for k in range(_NCHUNK):
        d1, d2 = descs[k]
        d1.wait()
        d2.wait()
        pltpu.sync_copy(embs[k], sums_sh.at[idxs[k]], add=True)
    plsc.subcore_barrier()

    r0 = sid * _ZROWS
    pltpu.sync_copy(sums_sh.at[pl.ds(r0, _ZROWS)],
                    psums_hbm.at[cid, pl.ds(r0, _ZROWS)])


@functools.cache
def _sc_segsum():
    return pl.kernel(
        _sc_segsum_body,
        out_type=jax.ShapeDtypeStruct((_NC, _CPAD, _D), jnp.float32),
        mesh=plsc.VectorSubcoreMesh(core_axis_name="c", subcore_axis_name="s",
                                    num_cores=_NC, num_subcores=_NS),
        scratch_types=(
            [pltpu.VMEM((_CHUNK, _D), jnp.float32)] * _NCHUNK
            + [pltpu.VMEM((_CHUNK,), jnp.int32)] * _NCHUNK
            + [pltpu.VMEM((_ZROWS, _D), jnp.float32),
               pltpu.VMEM_SHARED((_CPAD, _D), jnp.float32)]
            + [pltpu.SemaphoreType.DMA] * _NCHUNK
        ),
    )


_BLK = 2048
_NBLK = _B // _BLK
_MBLK = 4096
_NMBLK = _B // _MBLK


def _tc_counts_body(lab_ref, cnt_ref):
    i = pl.program_id(0)

    @pl.when(i == 0)
    def _():
        cnt_ref[...] = jnp.zeros((_CPAD, 1), jnp.float32)

    lab = lab_ref[0]
    oh = (jnp.broadcast_to(lab, (_CPAD, _BLK))
          == lax.broadcasted_iota(jnp.int32, (_CPAD, _BLK), 0))
    ones = jnp.ones((_BLK, 1), jnp.float32)
    cnt_ref[...] += lax.dot_general(oh.astype(jnp.float32), ones,
                                    (((1,), (0,)), ((), ())),
                                    preferred_element_type=jnp.float32)


_tc_counts = pl.pallas_call(
    _tc_counts_body,
    grid=(_NBLK,),
    in_specs=[pl.BlockSpec((1, 1, _BLK), lambda i: (i, 0, 0))],
    out_specs=pl.BlockSpec((_CPAD, 1), lambda i: (0, 0)),
    out_shape=jax.ShapeDtypeStruct((_CPAD, 1), jnp.float32),
)


def _tc_body(emb_ref, psums_ref, cnt_ref, proto_ref, init_ref,
             newp_ref, pred_ref, dist_ref):
    sums = psums_ref[0, :_C] + psums_ref[1, :_C]
    cnt = cnt_ref[...][:_C]
    cls_mean = sums / jnp.maximum(cnt, 1.0)
    present = cnt > 0.0
    initm = init_ref[...] > 0.0
    protos = proto_ref[...]
    ema = _MOM * protos + (1.0 - _MOM) * cls_mean
    upd = jnp.where(initm, ema, cls_mean)
    newp = jnp.where(present, upd, protos)
    newp_ref[...] = newp
    new_init = jnp.logical_or(initm, present)

    pn = jnp.sqrt(jnp.sum(newp * newp, axis=1, keepdims=True))
    pnorm = newp / jnp.maximum(pn, 1e-12)

    e = emb_ref[...]
    en = jnp.sqrt(jnp.sum(e * e, axis=1, keepdims=True))
    en_inv = e / jnp.maximum(en, 1e-12)

    simsT = lax.dot_general(pnorm, en_inv, (((1,), (1,)), ((), ())),
                            preferred_element_type=jnp.float32)
    simsT = jnp.where(new_init, simsT, -jnp.inf)
    m = jnp.max(simsT, axis=0, keepdims=True)
    row = lax.broadcasted_iota(jnp.int32, simsT.shape, 0)
    pred = jnp.min(jnp.where(simsT == m, row, _C), axis=0, keepdims=True)
    pred_ref[0] = pred
    dist_ref[0] = 1.0 - m


_tc_predict = pl.pallas_call(
    _tc_body,
    grid=(_NMBLK,),
    in_specs=[
        pl.BlockSpec((_MBLK, _D), lambda i: (i, 0)),
        pl.BlockSpec((_NC, _CPAD, _D), lambda i: (0, 0, 0)),
        pl.BlockSpec((_CPAD, 1), lambda i: (0, 0)),
        pl.BlockSpec((_C, _D), lambda i: (0, 0)),
        pl.BlockSpec((_C, 1), lambda i: (0, 0)),
    ],
    out_specs=[
        pl.BlockSpec((_C, _D), lambda i: (0, 0)),
        pl.BlockSpec((1, 1, _MBLK), lambda i: (i, 0, 0)),
        pl.BlockSpec((1, 1, _MBLK), lambda i: (i, 0, 0)),
    ],
    out_shape=[
        jax.ShapeDtypeStruct((_C, _D), jnp.float32),
        jax.ShapeDtypeStruct((_NMBLK, 1, _MBLK), jnp.int32),
        jax.ShapeDtypeStruct((_NMBLK, 1, _MBLK), jnp.float32),
    ],
)


def kernel(embeddings, labels, prototypes, initialized):
    psums = _sc_segsum()(embeddings, labels)
    lab3 = labels.reshape(_NBLK, 1, _BLK)
    cnts = _tc_counts(lab3)
    init_col = initialized.astype(jnp.float32).reshape(_C, 1)
    newp, pred2d, dist2d = _tc_predict(embeddings, psums, cnts,
                                       prototypes, init_col)
    return newp, pred2d.reshape(_B), dist2d.reshape(_B)
